# baseline (speedup 1.0000x reference)
import jax
import jax.numpy as jnp
from jax.experimental import pallas as pl
from jax.experimental.pallas import tpu as pltpu

DIM = 768
HID = 1024
E = 8
TOPK = 2

_NEG = -1e30


def _router_body(x_ref, gw_ref, bias_ref, c_ref):
    xt = x_ref[...]
    logits = jax.lax.dot_general(
        xt, gw_ref[...], (((1,), (1,)), ((), ())),
        preferred_element_type=jnp.float32)
    m = jnp.max(logits, axis=1, keepdims=True)
    ex = jnp.exp(logits - m)
    scores = ex / jnp.sum(ex, axis=1, keepdims=True)
    b = scores + bias_ref[...]
    iota = jax.lax.broadcasted_iota(jnp.int32, b.shape, 1)
    m1 = jnp.max(b, axis=1, keepdims=True)
    e1 = jnp.min(jnp.where(b >= m1, iota, E), axis=1, keepdims=True)
    b2 = jnp.where(iota == e1, _NEG, b)
    m2 = jnp.max(b2, axis=1, keepdims=True)
    e2 = jnp.min(jnp.where(b2 >= m2, iota, E), axis=1, keepdims=True)
    keep = (iota == e1) | (iota == e2)
    c_ref[...] = jnp.where(keep, scores, 0.0)


def _moe_body(c_ref, x_ref, w1_ref, w2_ref, w3_ref, o_ref):
    e = pl.program_id(0)
    c = c_ref[...]
    sel = jax.lax.broadcasted_iota(jnp.int32, c.shape, 1) == e
    ce = jnp.sum(jnp.where(sel, c, 0.0), axis=1, keepdims=True)
    xe = (x_ref[...] * ce).astype(jnp.bfloat16)
    h1 = jax.lax.dot_general(
        xe, w1_ref[0], (((1,), (0,)), ((), ())),
        preferred_element_type=jnp.float32)
    h3 = jax.lax.dot_general(
        xe, w3_ref[0], (((1,), (0,)), ((), ())),
        preferred_element_type=jnp.float32)
    h = ((h1 * jax.lax.logistic(h1)) * h3).astype(jnp.bfloat16)
    y = jax.lax.dot_general(
        h, w2_ref[0], (((1,), (0,)), ((), ())),
        preferred_element_type=jnp.float32)

    @pl.when(e == 0)
    def _init():
        o_ref[...] = y

    @pl.when(e > 0)
    def _acc():
        o_ref[...] += y


def kernel(x, gate_w, w1, w2, w3, expert_bias):
    bs, slen, dim = x.shape
    T = bs * slen
    xt = x.reshape(T, dim)

    c = pl.pallas_call(
        _router_body,
        out_shape=jax.ShapeDtypeStruct((T, E), jnp.float32),
    )(xt, gate_w, expert_bias.reshape(1, E))

    out = pl.pallas_call(
        _moe_body,
        grid=(E,),
        in_specs=[
            pl.BlockSpec((T, E), lambda e: (0, 0)),
            pl.BlockSpec((T, dim), lambda e: (0, 0)),
            pl.BlockSpec((1, dim, HID), lambda e: (e, 0, 0)),
            pl.BlockSpec((1, HID, dim), lambda e: (e, 0, 0)),
            pl.BlockSpec((1, dim, HID), lambda e: (e, 0, 0)),
        ],
        out_specs=pl.BlockSpec((T, dim), lambda e: (0, 0)),
        out_shape=jax.ShapeDtypeStruct((T, dim), jnp.float32),
        compiler_params=pltpu.CompilerParams(
            dimension_semantics=("arbitrary",),
        ),
    )(c, xt, w1.astype(jnp.bfloat16), w2.astype(jnp.bfloat16),
      w3.astype(jnp.bfloat16))

    return out.reshape(bs, slen, dim)


# speedup vs baseline: 1.3527x; 1.3527x over previous
import jax
import jax.numpy as jnp
from jax.experimental import pallas as pl
from jax.experimental.pallas import tpu as pltpu

DIM = 768
HID = 1024
E = 8
TOPK = 2

_NEG = -1e30


def _router_body(x_ref, gw_ref, bias_ref, c_ref):
    xt = x_ref[...]
    logits = jax.lax.dot_general(
        xt, gw_ref[...], (((1,), (1,)), ((), ())),
        preferred_element_type=jnp.float32)
    m = jnp.max(logits, axis=1, keepdims=True)
    ex = jnp.exp(logits - m)
    scores = ex / jnp.sum(ex, axis=1, keepdims=True)
    b = scores + bias_ref[...]
    iota = jax.lax.broadcasted_iota(jnp.int32, b.shape, 1)
    m1 = jnp.max(b, axis=1, keepdims=True)
    e1 = jnp.min(jnp.where(b >= m1, iota, E), axis=1, keepdims=True)
    b2 = jnp.where(iota == e1, _NEG, b)
    m2 = jnp.max(b2, axis=1, keepdims=True)
    e2 = jnp.min(jnp.where(b2 >= m2, iota, E), axis=1, keepdims=True)
    keep = (iota == e1) | (iota == e2)
    c_ref[...] = jnp.where(keep, scores, 0.0)


def _moe_body(c_ref, x_ref, w1_ref, w2_ref, w3_ref, o_ref):
    e = pl.program_id(0)
    c = c_ref[...]
    sel = jax.lax.broadcasted_iota(jnp.int32, c.shape, 1) == e
    ce = jnp.sum(jnp.where(sel, c, 0.0), axis=1, keepdims=True)
    xe = (x_ref[...] * ce).astype(jnp.bfloat16)
    h1 = jax.lax.dot_general(
        xe, w1_ref[0].astype(jnp.bfloat16), (((1,), (0,)), ((), ())),
        preferred_element_type=jnp.float32)
    h3 = jax.lax.dot_general(
        xe, w3_ref[0].astype(jnp.bfloat16), (((1,), (0,)), ((), ())),
        preferred_element_type=jnp.float32)
    h = ((h1 * jax.lax.logistic(h1)) * h3).astype(jnp.bfloat16)
    y = jax.lax.dot_general(
        h, w2_ref[0].astype(jnp.bfloat16), (((1,), (0,)), ((), ())),
        preferred_element_type=jnp.float32)

    @pl.when(e == 0)
    def _init():
        o_ref[...] = y

    @pl.when(e > 0)
    def _acc():
        o_ref[...] += y


def kernel(x, gate_w, w1, w2, w3, expert_bias):
    bs, slen, dim = x.shape
    T = bs * slen
    xt = x.reshape(T, dim)

    c = pl.pallas_call(
        _router_body,
        out_shape=jax.ShapeDtypeStruct((T, E), jnp.float32),
    )(xt, gate_w, expert_bias.reshape(1, E))

    out = pl.pallas_call(
        _moe_body,
        grid=(E,),
        in_specs=[
            pl.BlockSpec((T, E), lambda e: (0, 0)),
            pl.BlockSpec((T, dim), lambda e: (0, 0)),
            pl.BlockSpec((1, dim, HID), lambda e: (e, 0, 0)),
            pl.BlockSpec((1, HID, dim), lambda e: (e, 0, 0)),
            pl.BlockSpec((1, dim, HID), lambda e: (e, 0, 0)),
        ],
        out_specs=pl.BlockSpec((T, dim), lambda e: (0, 0)),
        out_shape=jax.ShapeDtypeStruct((T, dim), jnp.float32),
        compiler_params=pltpu.CompilerParams(
            dimension_semantics=("arbitrary",),
        ),
    )(c, xt, w1, w2, w3)

    return out.reshape(bs, slen, dim)
